# Initial kernel scaffold; baseline (speedup 1.0000x reference)
import jax
import jax.numpy as jnp
from jax.experimental import pallas as pl


def kernel(x, edge_index, edge_attr, start_idx, end_idx, x_1, x_2, params):
    raise NotImplementedError("write your pallas kernel here")


# speedup vs baseline: 1.1375x; 1.1375x over previous
import functools

import jax
import jax.numpy as jnp
from jax import lax
from jax.experimental import pallas as pl
from jax.experimental.pallas import tpu as pltpu

N_NODES = 10000
HID = 512
D_INV_SQRT = 1.0 / (512.0 ** 0.5)
EPS = 1e-16

BM = 400


def _mm_bias_body(x_ref, w_ref, b_ref, o_ref):
    o_ref[...] = (
        jnp.dot(x_ref[...], w_ref[...], preferred_element_type=jnp.float32)
        + b_ref[...]
    )


def _mm_bias(x, w, b, bm=BM):
    m, kdim = x.shape
    n = w.shape[1]
    return pl.pallas_call(
        _mm_bias_body,
        grid=(m // bm,),
        in_specs=[
            pl.BlockSpec((bm, kdim), lambda i: (i, 0)),
            pl.BlockSpec((kdim, n), lambda i: (0, 0)),
            pl.BlockSpec((1, n), lambda i: (0, 0)),
        ],
        out_specs=pl.BlockSpec((bm, n), lambda i: (i, 0)),
        out_shape=jax.ShapeDtypeStruct((m, n), jnp.float32),
    )(x, w, b.reshape(1, n))


def _gate_body(raw_ref, r_ref, d_ref, s2_ref, we_ref, ac_ref, bc_ref, o_ref):
    inv_d = 1.0 / (d_ref[...] + EPS)
    out = (raw_ref[...] + s2_ref[...] * we_ref[...]) * inv_d
    r = r_ref[...]
    bl = jnp.sum(out * ac_ref[...] + r * bc_ref[...], axis=1, keepdims=True)
    beta = jax.nn.sigmoid(bl)
    o_ref[...] = jnp.maximum(beta * r + (1.0 - beta) * out, 0.0)


def _gate(out_raw, r, denom, s2, we_row, w_ac, w_bc):
    n = out_raw.shape[0]
    vspec = pl.BlockSpec((1, HID), lambda i: (0, 0))
    return pl.pallas_call(
        _gate_body,
        grid=(n // BM,),
        in_specs=[
            pl.BlockSpec((BM, HID), lambda i: (i, 0)),
            pl.BlockSpec((BM, HID), lambda i: (i, 0)),
            pl.BlockSpec((BM, 1), lambda i: (i, 0)),
            pl.BlockSpec((BM, 1), lambda i: (i, 0)),
            vspec, vspec, vspec,
        ],
        out_specs=pl.BlockSpec((BM, HID), lambda i: (i, 0)),
        out_shape=jax.ShapeDtypeStruct((n, HID), jnp.float32),
    )(out_raw, r, denom.reshape(n, 1), s2.reshape(n, 1),
      we_row.reshape(1, HID), w_ac.reshape(1, HID), w_bc.reshape(1, HID))


def _mlp_body(f_ref, w1_ref, b1_ref, g_ref, be_ref, w2_ref, b2_ref,
              wh_ref, bh_ref, o_ref):
    h = jnp.dot(f_ref[...], w1_ref[...], preferred_element_type=jnp.float32)
    h = jnp.maximum(h + b1_ref[...], 0.0)
    mu = jnp.mean(h, axis=1, keepdims=True)
    var = jnp.mean((h - mu) ** 2, axis=1, keepdims=True)
    h = (h - mu) * jax.lax.rsqrt(var + 1e-5) * g_ref[...] + be_ref[...]
    h = jnp.dot(h, w2_ref[...], preferred_element_type=jnp.float32)
    h = jnp.maximum(h + b2_ref[...], 0.0)
    o_ref[...] = (
        jnp.dot(h, wh_ref[...], preferred_element_type=jnp.float32)
        + bh_ref[...]
    )


def _mlp(feats, mp):
    b = feats.shape[0]
    bm = 512
    d1 = mp["W1"].shape[0]
    d2 = mp["W1"].shape[1]
    d3 = mp["W2"].shape[1]
    return pl.pallas_call(
        _mlp_body,
        grid=(b // bm,),
        in_specs=[
            pl.BlockSpec((bm, d1), lambda i: (i, 0)),
            pl.BlockSpec((d1, d2), lambda i: (0, 0)),
            pl.BlockSpec((1, d2), lambda i: (0, 0)),
            pl.BlockSpec((1, d2), lambda i: (0, 0)),
            pl.BlockSpec((1, d2), lambda i: (0, 0)),
            pl.BlockSpec((d2, d3), lambda i: (0, 0)),
            pl.BlockSpec((1, d3), lambda i: (0, 0)),
            pl.BlockSpec((d3, 1), lambda i: (0, 0)),
            pl.BlockSpec((1, 1), lambda i: (0, 0)),
        ],
        out_specs=pl.BlockSpec((bm, 1), lambda i: (i, 0)),
        out_shape=jax.ShapeDtypeStruct((b, 1), jnp.float32),
    )(feats, mp["W1"], mp["b1"].reshape(1, d2), mp["ln_g"].reshape(1, d2),
      mp["ln_b"].reshape(1, d2), mp["W2"], mp["b2"].reshape(1, d3),
      mp["Wh"], mp["bh"].reshape(1, 1))


def _edge_phase(q, k, v, q_we, src, dst, ea):
    logits = (jnp.sum(q[dst] * k[src], axis=-1) + ea * q_we[dst]) * D_INV_SQRT
    w = jnp.exp(logits)
    denom = jax.ops.segment_sum(w, dst, num_segments=N_NODES)
    s2 = jax.ops.segment_sum(w * ea, dst, num_segments=N_NODES)
    out_raw = jax.ops.segment_sum(w[:, None] * v[src], dst,
                                  num_segments=N_NODES)
    return out_raw, denom, s2


def _layer(h, src, dst, ea, p, pad_k):
    n = h.shape[0]
    din = h.shape[1]
    if pad_k:
        hp = jnp.pad(h, ((0, 0), (0, pad_k)))
        wcat = jnp.pad(
            jnp.concatenate([p["Wq"], p["Wk"], p["Wv"], p["Wskip"]], axis=1),
            ((0, pad_k), (0, 0)))
    else:
        hp = h
        wcat = jnp.concatenate([p["Wq"], p["Wk"], p["Wv"], p["Wskip"]], axis=1)
    bcat = jnp.concatenate([p["bq"], p["bk"], p["bv"], p["bskip"]])
    qkvr = _mm_bias(hp, wcat, bcat)
    q = qkvr[:, :HID]
    k = qkvr[:, HID:2 * HID]
    v = qkvr[:, 2 * HID:3 * HID]
    r = qkvr[:, 3 * HID:]

    we_row = p["We"][0]
    q_we = q @ we_row
    out_raw, denom, s2 = _edge_phase(q, k, v, q_we, src, dst, ea)

    wb = p["Wbeta"][:, 0]
    w_ac = wb[:HID] + wb[2 * HID:]
    w_bc = wb[HID:2 * HID] - wb[2 * HID:]
    return _gate(out_raw, r, denom, s2, we_row, w_ac, w_bc)


def kernel(x, edge_index, edge_attr, start_idx, end_idx, x_1, x_2, params):
    src = edge_index[0]
    dst = edge_index[1]
    ea = edge_attr[:, 0]

    h = _layer(x, src, dst, ea, params["conv0"], pad_k=26)
    h = _layer(h, src, dst, ea, params["conv1"], pad_k=0)
    h = _layer(h, src, dst, ea, params["conv2"], pad_k=0)

    feats = jnp.concatenate([h[start_idx], h[end_idx], h[x_1], h[x_2]], axis=1)
    return _mlp(feats, params["mlp"])
